# Initial kernel scaffold; baseline (speedup 1.0000x reference)
import jax
import jax.numpy as jnp
from jax.experimental import pallas as pl


def kernel(inputs, coeff, idx):
    raise NotImplementedError("write your pallas kernel here")


# speedup vs baseline: 4.8577x; 4.8577x over previous
import functools

import jax
import jax.numpy as jnp
from jax import lax
from jax.experimental import pallas as pl
from jax.experimental.pallas import tpu as pltpu
from jax.experimental.pallas import tpu_sc as plsc

_NC = 2
_NS = 16
_NW = _NC * _NS
_LANES = 16
_CHUNK = 128


def _relu_body(x_ref, o_ref):
    o_ref[...] = jnp.maximum(x_ref[...], 0.0)


@functools.cache
def _dense_relu(Bn, Dn, block_rows):
    return pl.pallas_call(
        _relu_body,
        grid=(Bn // block_rows,),
        in_specs=[pl.BlockSpec((block_rows, Dn), lambda i: (i, 0))],
        out_specs=pl.BlockSpec((block_rows, Dn), lambda i: (i, 0)),
        out_shape=jax.ShapeDtypeStruct((Bn, Dn), jnp.float32),
    )


@functools.cache
def _sc_fixup(Bn, Dn):
    b_per_w = Bn // _NW
    n_chunks = b_per_w // _CHUNK

    @functools.partial(
        pl.kernel,
        mesh=plsc.VectorSubcoreMesh(core_axis_name="c", subcore_axis_name="s"),
        out_type=(),
        scratch_types=[
            pltpu.VMEM((_CHUNK,), jnp.int32),
            pltpu.VMEM((_CHUNK,), jnp.float32),
            pltpu.VMEM((_LANES,), jnp.float32),
            pltpu.SemaphoreType.DMA,
        ],
    )
    def fix(x_hbm, idx_hbm, scale_hbm, y_ref, idxv, vals, sv, sem):
        wid = lax.axis_index("s") * _NC + lax.axis_index("c")
        base = wid * b_per_w
        pltpu.sync_copy(scale_hbm, sv)
        scale = sv[...]
        for c in range(n_chunks):
            cb = base + c * _CHUNK
            pltpu.sync_copy(idx_hbm.at[pl.ds(cb, _CHUNK)], idxv)
            for j in range(_CHUNK // _LANES):
                rows = lax.iota(jnp.int32, _LANES) + (cb + j * _LANES)
                sl = pl.ds(j * _LANES, _LANES)
                idxv[sl] = idxv[sl] + rows * Dn
            pltpu.async_copy(x_hbm.at[idxv], vals, sem).wait()
            for j in range(_CHUNK // _LANES):
                sl = pl.ds(j * _LANES, _LANES)
                vals[sl] = jnp.maximum(vals[sl] * scale, 0.0)
            pltpu.async_copy(vals, y_ref.at[idxv], sem).wait()

    return fix


def kernel(inputs, coeff, idx):
    Bn, Dn = inputs.shape
    y = _dense_relu(Bn, Dn, 512)(inputs)
    scale = jnp.full((_LANES,), 1.0 - 2.0 * coeff, dtype=jnp.float32)
    y_ref = jax.new_ref(y.reshape(-1))
    _sc_fixup(Bn, Dn)(inputs.reshape(-1), idx, scale, y_ref)
    return y_ref[...].reshape(Bn, Dn)


# speedup vs baseline: 9.9820x; 2.0549x over previous
import functools

import jax
import jax.numpy as jnp
from jax import lax
from jax.experimental import pallas as pl
from jax.experimental.pallas import tpu as pltpu
from jax.experimental.pallas import tpu_sc as plsc

_NC = 2
_NS = 16
_NW = _NC * _NS
_LANES = 16
_CHUNK = 128


@functools.cache
def _sc_gather(Bn, Dn):
    b_per_w = Bn // _NW
    n_chunks = b_per_w // _CHUNK

    @functools.partial(
        pl.kernel,
        mesh=plsc.VectorSubcoreMesh(core_axis_name="c", subcore_axis_name="s"),
        out_type=jax.ShapeDtypeStruct((Bn,), jnp.float32),
        scratch_types=[
            pltpu.VMEM((_CHUNK,), jnp.int32),
            pltpu.VMEM((_CHUNK,), jnp.float32),
            pltpu.VMEM((_LANES,), jnp.float32),
            pltpu.SemaphoreType.DMA,
        ],
    )
    def gather(x_hbm, idx_hbm, scale_hbm, v_hbm, idxv, vals, sv, sem):
        wid = lax.axis_index("s") * _NC + lax.axis_index("c")
        base = wid * b_per_w
        pltpu.sync_copy(scale_hbm, sv)
        scale = sv[...]
        for c in range(n_chunks):
            cb = base + c * _CHUNK
            pltpu.sync_copy(idx_hbm.at[pl.ds(cb, _CHUNK)], idxv)
            for j in range(_CHUNK // _LANES):
                rows = lax.iota(jnp.int32, _LANES) + (cb + j * _LANES)
                sl = pl.ds(j * _LANES, _LANES)
                idxv[sl] = idxv[sl] + rows * Dn
            pltpu.async_copy(x_hbm.at[idxv], vals, sem).wait()
            for j in range(_CHUNK // _LANES):
                sl = pl.ds(j * _LANES, _LANES)
                vals[sl] = jnp.maximum(vals[sl] * scale, 0.0)
            pltpu.sync_copy(vals, v_hbm.at[pl.ds(cb, _CHUNK)])

    return gather


def _merge_body(x_ref, idx_ref, v_ref, o_ref):
    x = x_ref[...]
    idx = idx_ref[0, 0, :]
    v = v_ref[0, 0, :]
    col = lax.broadcasted_iota(jnp.int32, x.shape, 1)
    mask = col == idx[:, None]
    o_ref[...] = jnp.where(mask, v[:, None], jnp.maximum(x, 0.0))


@functools.cache
def _dense_merge(Bn, Dn, block_rows):
    nb = Bn // block_rows
    return pl.pallas_call(
        _merge_body,
        grid=(nb,),
        in_specs=[
            pl.BlockSpec((block_rows, Dn), lambda i: (i, 0)),
            pl.BlockSpec((1, 1, block_rows), lambda i: (i, 0, 0)),
            pl.BlockSpec((1, 1, block_rows), lambda i: (i, 0, 0)),
        ],
        out_specs=pl.BlockSpec((block_rows, Dn), lambda i: (i, 0)),
        out_shape=jax.ShapeDtypeStruct((Bn, Dn), jnp.float32),
    )


def kernel(inputs, coeff, idx):
    Bn, Dn = inputs.shape
    block_rows = 512
    nb = Bn // block_rows
    scale = jnp.full((_LANES,), 1.0 - 2.0 * coeff, dtype=jnp.float32)
    v = _sc_gather(Bn, Dn)(inputs.reshape(-1), idx, scale)
    idx3 = idx.reshape(nb, 1, block_rows)
    v3 = v.reshape(nb, 1, block_rows)
    return _dense_merge(Bn, Dn, block_rows)(inputs, idx3, v3)


# speedup vs baseline: 13.8837x; 1.3909x over previous
import functools

import jax
import jax.numpy as jnp
from jax import lax
from jax.experimental import pallas as pl
from jax.experimental.pallas import tpu as pltpu
from jax.experimental.pallas import tpu_sc as plsc

_NC = 2
_NS = 16
_NW = _NC * _NS
_LANES = 16


_ROUND = 64


@functools.cache
def _sc_gather(Bn, Dn):
    b_per_w = Bn // _NW
    n_rounds = b_per_w // _ROUND

    @functools.partial(
        pl.kernel,
        mesh=plsc.VectorSubcoreMesh(core_axis_name="c", subcore_axis_name="s"),
        out_type=jax.ShapeDtypeStruct((Bn,), jnp.float32),
        compiler_params=pltpu.CompilerParams(
            use_tc_tiling_on_sc=True, needs_layout_passes=False
        ),
        scratch_types=[
            pltpu.VMEM((b_per_w,), jnp.int32),
            pltpu.VMEM((_ROUND, 8, 128), jnp.float32),
            pltpu.VMEM((b_per_w,), jnp.float32),
            pltpu.VMEM((_LANES,), jnp.float32),
            pltpu.SemaphoreType.DMA,
        ],
    )
    def gather(x_hbm, idx_hbm, scale_hbm, v_hbm, idx_v, bufs, vals, sv, sem):
        wid = lax.axis_index("s") * _NC + lax.axis_index("c")
        base = wid * b_per_w
        pltpu.sync_copy(scale_hbm, sv)
        scale = sv[...]
        pltpu.sync_copy(idx_hbm.at[pl.ds(base, b_per_w)], idx_v)
        lanes_i = lax.iota(jnp.int32, _LANES)
        sub = lanes_i & 7
        for rd in range(n_rounds):
            r0 = rd * _ROUND
            handles = []
            for g in range(_ROUND // _LANES):
                cvec = (idx_v[pl.ds(r0 + g * _LANES, _LANES)] >> 7) << 7
                for j in range(_LANES):
                    i = g * _LANES + j
                    t = r0 + i
                    c0 = pl.multiple_of(cvec[j], 128)
                    stripe = base + ((t >> 3) << 3)
                    h = pltpu.async_copy(
                        x_hbm.at[pl.ds(stripe, 8), pl.ds(c0, 128)],
                        bufs.at[i],
                        sem,
                    )
                    handles.append(h)
            for h in handles:
                h.wait()
            for g in range(_ROUND // _LANES):
                sl = pl.ds(r0 + g * _LANES, _LANES)
                lane = idx_v[sl] & 127
                tidx = lanes_i + g * _LANES
                x_sel = plsc.load_gather(bufs, [tidx, sub, lane])
                vals[sl] = jnp.maximum(x_sel * scale, 0.0)
        pltpu.sync_copy(vals, v_hbm.at[pl.ds(base, b_per_w)])

    return gather


def _merge_body(x_ref, idx_ref, v_ref, o_ref):
    x = x_ref[...]
    idx = idx_ref[0, 0, :]
    v = v_ref[0, 0, :]
    col = lax.broadcasted_iota(jnp.int32, x.shape, 1)
    mask = col == idx[:, None]
    o_ref[...] = jnp.where(mask, v[:, None], jnp.maximum(x, 0.0))


@functools.cache
def _dense_merge(Bn, Dn, block_rows):
    nb = Bn // block_rows
    return pl.pallas_call(
        _merge_body,
        grid=(nb,),
        in_specs=[
            pl.BlockSpec((block_rows, Dn), lambda i: (i, 0)),
            pl.BlockSpec((1, 1, block_rows), lambda i: (i, 0, 0)),
            pl.BlockSpec((1, 1, block_rows), lambda i: (i, 0, 0)),
        ],
        out_specs=pl.BlockSpec((block_rows, Dn), lambda i: (i, 0)),
        out_shape=jax.ShapeDtypeStruct((Bn, Dn), jnp.float32),
    )


def kernel(inputs, coeff, idx):
    Bn, Dn = inputs.shape
    block_rows = 512
    nb = Bn // block_rows
    scale = jnp.full((_LANES,), 1.0 - 2.0 * coeff, dtype=jnp.float32)
    v = _sc_gather(Bn, Dn)(inputs, idx, scale)
    idx3 = idx.reshape(nb, 1, block_rows)
    v3 = v.reshape(nb, 1, block_rows)
    return _dense_merge(Bn, Dn, block_rows)(inputs, idx3, v3)


# speedup vs baseline: 14.1393x; 1.0184x over previous
import functools

import jax
import jax.numpy as jnp
from jax import lax
from jax.experimental import pallas as pl
from jax.experimental.pallas import tpu as pltpu
from jax.experimental.pallas import tpu_sc as plsc

_NC = 2
_NS = 16
_NW = _NC * _NS
_LANES = 16


_ROUND = 64


@functools.cache
def _sc_gather(Bn, Dn, rows, off):
    b_per_w = rows // _NW
    n_rounds = b_per_w // _ROUND

    @functools.partial(
        pl.kernel,
        mesh=plsc.VectorSubcoreMesh(core_axis_name="c", subcore_axis_name="s"),
        out_type=jax.ShapeDtypeStruct((rows,), jnp.float32),
        compiler_params=pltpu.CompilerParams(
            use_tc_tiling_on_sc=True, needs_layout_passes=False
        ),
        scratch_types=[
            pltpu.VMEM((b_per_w,), jnp.int32),
            pltpu.VMEM((_ROUND, 8, 128), jnp.float32),
            pltpu.VMEM((b_per_w,), jnp.float32),
            pltpu.VMEM((_LANES,), jnp.float32),
            pltpu.SemaphoreType.DMA,
        ],
    )
    def gather(x_hbm, idx_hbm, scale_hbm, v_hbm, idx_v, bufs, vals, sv, sem):
        wid = lax.axis_index("s") * _NC + lax.axis_index("c")
        base = wid * b_per_w
        pltpu.sync_copy(scale_hbm, sv)
        scale = sv[...]
        pltpu.sync_copy(idx_hbm.at[pl.ds(base, b_per_w)], idx_v)
        lanes_i = lax.iota(jnp.int32, _LANES)
        sub = lanes_i & 7
        for rd in range(n_rounds):
            r0 = rd * _ROUND
            handles = []
            for g in range(_ROUND // _LANES):
                cvec = (idx_v[pl.ds(r0 + g * _LANES, _LANES)] >> 7) << 7
                for j in range(_LANES):
                    i = g * _LANES + j
                    t = r0 + i
                    c0 = pl.multiple_of(cvec[j], 128)
                    stripe = off + base + ((t >> 3) << 3)
                    h = pltpu.async_copy(
                        x_hbm.at[pl.ds(stripe, 8), pl.ds(c0, 128)],
                        bufs.at[i],
                        sem,
                    )
                    handles.append(h)
            for h in handles:
                h.wait()
            for g in range(_ROUND // _LANES):
                sl = pl.ds(r0 + g * _LANES, _LANES)
                lane = idx_v[sl] & 127
                tidx = lanes_i + g * _LANES
                x_sel = plsc.load_gather(bufs, [tidx, sub, lane])
                vals[sl] = jnp.maximum(x_sel * scale, 0.0)
        pltpu.sync_copy(vals, v_hbm.at[pl.ds(base, b_per_w)])

    return gather


def _merge_body(x_ref, idx_ref, v_ref, o_ref):
    x = x_ref[...]
    idx = idx_ref[0, 0, :]
    v = v_ref[0, 0, :]
    col = lax.broadcasted_iota(jnp.int32, x.shape, 1)
    mask = col == idx[:, None]
    o_ref[...] = jnp.where(mask, v[:, None], jnp.maximum(x, 0.0))


def _merge_body_acc(x_ref, idx_ref, v_ref, y_ref, o_ref):
    del y_ref
    _merge_body(x_ref, idx_ref, v_ref, o_ref)


@functools.cache
def _dense_merge_chunk(Bn, Dn, block_rows, chunk_blocks, blk_off, aliased):
    x_spec = pl.BlockSpec((block_rows, Dn), lambda i: (i + blk_off, 0))
    s_spec = pl.BlockSpec((1, 1, block_rows), lambda i: (i, 0, 0))
    in_specs = [x_spec, s_spec, s_spec]
    body = _merge_body
    io_aliases = {}
    if aliased:
        in_specs.append(pl.BlockSpec(memory_space=pl.ANY))
        body = _merge_body_acc
        io_aliases = {3: 0}
    return pl.pallas_call(
        body,
        grid=(chunk_blocks,),
        in_specs=in_specs,
        out_specs=pl.BlockSpec((block_rows, Dn), lambda i: (i + blk_off, 0)),
        out_shape=jax.ShapeDtypeStruct((Bn, Dn), jnp.float32),
        input_output_aliases=io_aliases,
    )


_NCHUNK = 2


def kernel(inputs, coeff, idx):
    Bn, Dn = inputs.shape
    block_rows = 512
    rows = Bn // _NCHUNK
    nbc = rows // block_rows
    scale = jnp.full((_LANES,), 1.0 - 2.0 * coeff, dtype=jnp.float32)
    vs = [
        _sc_gather(Bn, Dn, rows, k * rows)(
            inputs, lax.slice(idx, (k * rows,), ((k + 1) * rows,)), scale
        )
        for k in range(_NCHUNK)
    ]
    y = None
    for k in range(_NCHUNK):
        idx3 = lax.slice(idx, (k * rows,), ((k + 1) * rows,)).reshape(
            nbc, 1, block_rows
        )
        v3 = vs[k].reshape(nbc, 1, block_rows)
        fn = _dense_merge_chunk(Bn, Dn, block_rows, nbc, k * nbc, y is not None)
        args = (inputs, idx3, v3) if y is None else (inputs, idx3, v3, y)
        y = fn(*args)
    return y


# speedup vs baseline: 15.7232x; 1.1120x over previous
import functools

import jax
import jax.numpy as jnp
from jax import lax
from jax.experimental import pallas as pl
from jax.experimental.pallas import tpu as pltpu
from jax.experimental.pallas import tpu_sc as plsc

_NC = 2
_NS = 16
_NW = _NC * _NS
_LANES = 16


_ROUND = 64


@functools.cache
def _sc_gather(Bn, Dn, rows, off):
    b_per_w = rows // _NW
    n_rounds = b_per_w // _ROUND

    @functools.partial(
        pl.kernel,
        mesh=plsc.VectorSubcoreMesh(core_axis_name="c", subcore_axis_name="s"),
        out_type=jax.ShapeDtypeStruct((rows,), jnp.float32),
        compiler_params=pltpu.CompilerParams(
            use_tc_tiling_on_sc=True, needs_layout_passes=False
        ),
        scratch_types=[
            pltpu.VMEM((b_per_w,), jnp.int32),
            pltpu.VMEM((_ROUND, _LANES), jnp.float32),
            pltpu.VMEM((b_per_w,), jnp.float32),
            pltpu.VMEM((_LANES,), jnp.float32),
            pltpu.SemaphoreType.DMA,
        ],
    )
    def gather(x_hbm, idx_hbm, scale_hbm, v_hbm, idx_v, bufs, vals, sv, sem):
        wid = lax.axis_index("s") * _NC + lax.axis_index("c")
        base = wid * b_per_w
        pltpu.sync_copy(scale_hbm, sv)
        scale = sv[...]
        pltpu.sync_copy(idx_hbm.at[pl.ds(base, b_per_w)], idx_v)
        lanes_i = lax.iota(jnp.int32, _LANES)
        for rd in range(n_rounds):
            r0 = rd * _ROUND
            handles = []
            for g in range(_ROUND // _LANES):
                cvec = (idx_v[pl.ds(r0 + g * _LANES, _LANES)] >> 4) << 4
                for j in range(_LANES):
                    i = g * _LANES + j
                    t = r0 + i
                    c0 = pl.multiple_of(cvec[j], 16)
                    row = off + base + t
                    h = pltpu.async_copy(
                        x_hbm.at[row, pl.ds(c0, _LANES)],
                        bufs.at[i],
                        sem,
                    )
                    handles.append(h)
            for h in handles:
                h.wait()
            for g in range(_ROUND // _LANES):
                sl = pl.ds(r0 + g * _LANES, _LANES)
                lane = idx_v[sl] & (_LANES - 1)
                tidx = lanes_i + g * _LANES
                x_sel = plsc.load_gather(bufs, [tidx, lane])
                vals[sl] = jnp.maximum(x_sel * scale, 0.0)
        pltpu.sync_copy(vals, v_hbm.at[pl.ds(base, b_per_w)])

    return gather


def _merge_body(x_ref, idx_ref, v_ref, o_ref):
    x = x_ref[...]
    idx = idx_ref[0, 0, :]
    v = v_ref[0, 0, :]
    col = lax.broadcasted_iota(jnp.int32, x.shape, 1)
    mask = col == idx[:, None]
    o_ref[...] = jnp.where(mask, v[:, None], jnp.maximum(x, 0.0))


def _merge_body_acc(x_ref, idx_ref, v_ref, y_ref, o_ref):
    del y_ref
    _merge_body(x_ref, idx_ref, v_ref, o_ref)


@functools.cache
def _dense_merge_chunk(Bn, Dn, block_rows, chunk_blocks, blk_off, aliased):
    x_spec = pl.BlockSpec((block_rows, Dn), lambda i: (i + blk_off, 0))
    s_spec = pl.BlockSpec((1, 1, block_rows), lambda i: (i, 0, 0))
    in_specs = [x_spec, s_spec, s_spec]
    body = _merge_body
    io_aliases = {}
    if aliased:
        in_specs.append(pl.BlockSpec(memory_space=pl.ANY))
        body = _merge_body_acc
        io_aliases = {3: 0}
    return pl.pallas_call(
        body,
        grid=(chunk_blocks,),
        in_specs=in_specs,
        out_specs=pl.BlockSpec((block_rows, Dn), lambda i: (i + blk_off, 0)),
        out_shape=jax.ShapeDtypeStruct((Bn, Dn), jnp.float32),
        input_output_aliases=io_aliases,
    )


_NCHUNK = 2


def kernel(inputs, coeff, idx):
    Bn, Dn = inputs.shape
    block_rows = 512
    rows = Bn // _NCHUNK
    nbc = rows // block_rows
    scale = jnp.full((_LANES,), 1.0 - 2.0 * coeff, dtype=jnp.float32)
    vs = [
        _sc_gather(Bn, Dn, rows, k * rows)(
            inputs, lax.slice(idx, (k * rows,), ((k + 1) * rows,)), scale
        )
        for k in range(_NCHUNK)
    ]
    y = None
    for k in range(_NCHUNK):
        idx3 = lax.slice(idx, (k * rows,), ((k + 1) * rows,)).reshape(
            nbc, 1, block_rows
        )
        v3 = vs[k].reshape(nbc, 1, block_rows)
        fn = _dense_merge_chunk(Bn, Dn, block_rows, nbc, k * nbc, y is not None)
        args = (inputs, idx3, v3) if y is None else (inputs, idx3, v3, y)
        y = fn(*args)
    return y
